# baseline (speedup 1.0000x reference)
import functools

import jax
import jax.numpy as jnp
from jax import lax
from jax.experimental import pallas as pl
from jax.experimental.pallas import tpu as pltpu
from jax.experimental.pallas import tpu_sc as plsc

N, E, D, FD = 10000, 320000, 128, 64
NC, NS = 2, 16
NW = NC * NS
EPW = E // NW
NP = 10240
NPT = NP // NS
CW = 16
HD = D // 2
EPT = E // NS

C1 = 80
CG = 40
GKG = 5
NGG = EPW // (CG * GKG)
CS = 40
GKS = 5
NGS = EPT // (CS * GKS)


def _gather_body(x_hbm, tgt_hbm, src_hbm, tf_hbm, sf_hbm,
                 idx_all, rows_t, rows_s, gsem, wsem0, wsem1):
    wid = lax.axis_index("s") * NC + lax.axis_index("c")
    base = wid * EPW
    pltpu.sync_copy(tgt_hbm.at[pl.ds(base, EPW)], idx_all.at[0])
    pltpu.sync_copy(src_hbm.at[pl.ds(base, EPW)], idx_all.at[1])

    def phase(g, s):
        wsem = wsem0 if s == 0 else wsem1
        @pl.when(g >= 2)
        def _():
            offp = base + (g - 2) * GKG * CG
            for j in range(GKG):
                pltpu.make_async_copy(
                    rows_t.at[s, j], tf_hbm.at[pl.ds(offp + j * CG, CG)],
                    wsem).wait()
                pltpu.make_async_copy(
                    rows_s.at[s, j], sf_hbm.at[pl.ds(offp + j * CG, CG)],
                    wsem).wait()

        goff = g * GKG * CG
        handles = []
        for j in range(GKG):
            off = goff + j * CG
            handles.append(pltpu.async_copy(
                x_hbm.at[idx_all.at[0, pl.ds(off, CG)]], rows_t.at[s, j], gsem))
            handles.append(pltpu.async_copy(
                x_hbm.at[idx_all.at[1, pl.ds(off, CG)]], rows_s.at[s, j], gsem))
        for h in handles:
            h.wait()
        for j in range(GKG):
            off = base + goff + j * CG
            pltpu.async_copy(rows_t.at[s, j], tf_hbm.at[pl.ds(off, CG)], wsem)
            pltpu.async_copy(rows_s.at[s, j], sf_hbm.at[pl.ds(off, CG)], wsem)

    def body(h, carry):
        phase(2 * h, 0)
        phase(2 * h + 1, 1)
        return carry

    lax.fori_loop(0, NGG // 2, body, 0)
    for g, s in ((NGG - 2, 0), (NGG - 1, 1)):
        wsem = wsem0 if s == 0 else wsem1
        offp = base + g * GKG * CG
        for j in range(GKG):
            pltpu.make_async_copy(
                rows_t.at[s, j], tf_hbm.at[pl.ds(offp + j * CG, CG)],
                wsem).wait()
            pltpu.make_async_copy(
                rows_s.at[s, j], sf_hbm.at[pl.ds(offp + j * CG, CG)],
                wsem).wait()


def _scatter_body(mlo_hbm, mhi_hbm, tgt3d_hbm, zero_agg_hbm, zero_cnt_hbm,
                  ones_hbm, agg_hbm, cnt_hbm,
                  idx2d, rows_v, ones_v, wb_v, wbc_v, agg_sh, cnt_sh,
                  lsem, ssem0, ssem1):
    cid = lax.axis_index("c")
    sid = lax.axis_index("s")
    pltpu.sync_copy(zero_agg_hbm, agg_sh.at[pl.ds(sid * NPT, NPT)])
    pltpu.sync_copy(zero_cnt_hbm, cnt_sh.at[pl.ds(sid * NPT, NPT)])
    pltpu.sync_copy(ones_hbm, ones_v)
    pltpu.sync_copy(tgt3d_hbm.at[sid], idx2d)
    plsc.subcore_barrier()

    base = sid * EPT

    def drain_scatters(s):
        ssem = ssem0 if s == 0 else ssem1
        for j in range(GKS):
            pltpu.make_async_copy(
                rows_v.at[s, j], agg_sh.at[pl.ds(0, CS)], ssem).wait()

            @pl.when(cid == 0)
            def _():
                pltpu.make_async_copy(
                    ones_v, cnt_sh.at[pl.ds(0, CS)], ssem).wait()

    def phase(g, s):
        @pl.when(g >= 2)
        def _():
            drain_scatters(s)

        goff = g * GKS * CS

        @pl.when(cid == 0)
        def _():
            handles = []
            for j in range(GKS):
                off = base + goff + j * CS
                handles.append(pltpu.async_copy(
                    mlo_hbm.at[pl.ds(off, CS)], rows_v.at[s, j], lsem))
            for h in handles:
                h.wait()

        @pl.when(cid == 1)
        def _():
            handles = []
            for j in range(GKS):
                off = base + goff + j * CS
                handles.append(pltpu.async_copy(
                    mhi_hbm.at[pl.ds(off, CS)], rows_v.at[s, j], lsem))
            for h in handles:
                h.wait()
        ssem = ssem0 if s == 0 else ssem1
        for j in range(GKS):
            pltpu.async_copy(
                rows_v.at[s, j], agg_sh.at[idx2d.at[g * GKS + j]], ssem,
                add=True)

            @pl.when(cid == 0)
            def _():
                pltpu.async_copy(
                    ones_v, cnt_sh.at[idx2d.at[g * GKS + j]], ssem,
                    add=True)

    def body(h, carry):
        phase(2 * h, 0)
        phase(2 * h + 1, 1)
        return carry

    lax.fori_loop(0, NGS // 2, body, 0)
    drain_scatters(0)
    drain_scatters(1)
    plsc.subcore_barrier()
    for k in range(4):
        q = NPT // 4
        r0 = sid * NPT + k * q
        pltpu.sync_copy(agg_sh.at[pl.ds(r0, q)], wb_v)
        pltpu.sync_copy(wb_v, agg_hbm.at[cid, pl.ds(r0, q)])
    for k in range(2):
        q = NPT // 2
        r0 = sid * NPT + k * q
        pltpu.sync_copy(cnt_sh.at[pl.ds(r0, q)], wbc_v)
        pltpu.sync_copy(wbc_v, cnt_hbm.at[cid, pl.ds(r0, q)])


def _dense_body(tf_ref, sf_ref, attn_ref, g1a_ref, g1b_ref, g2a_ref, g2b_ref,
                b1a_ref, b1b_ref, b2a_ref, b2b_ref,
                w1a_ref, w1b_ref, w2a_ref, w2b_ref, bias_ref,
                mlo_ref, mhi_ref):
    def unpack(p):
        u = lax.bitcast_convert_type(p, jnp.uint32)
        lo = lax.bitcast_convert_type(u << 16, jnp.float32)
        hi = lax.bitcast_convert_type(u & jnp.uint32(0xFFFF0000), jnp.float32)
        return lo, hi
    tl, th = unpack(tf_ref[...])
    sl, sh = unpack(sf_ref[...])
    ssum = (jnp.sum(tl, axis=1, keepdims=True)
            + jnp.sum(th, axis=1, keepdims=True)
            + jnp.sum(sl, axis=1, keepdims=True)
            + jnp.sum(sh, axis=1, keepdims=True))
    sq = (jnp.sum(tl * tl, axis=1, keepdims=True)
          + jnp.sum(th * th, axis=1, keepdims=True)
          + jnp.sum(sl * sl, axis=1, keepdims=True)
          + jnp.sum(sh * sh, axis=1, keepdims=True))
    mu = ssum * (1.0 / (2 * D))
    var = sq * (1.0 / (2 * D)) - mu * mu
    inv = lax.rsqrt(var + 1e-5)
    h1 = jnp.maximum((tl - mu) * inv * g1a_ref[...] + b1a_ref[...], 0.0)
    h2 = jnp.maximum((th - mu) * inv * g1b_ref[...] + b1b_ref[...], 0.0)
    h3 = jnp.maximum((sl - mu) * inv * g2a_ref[...] + b2a_ref[...], 0.0)
    h4 = jnp.maximum((sh - mu) * inv * g2b_ref[...] + b2b_ref[...], 0.0)
    f32 = jnp.float32
    z = (jnp.dot(h1, w1a_ref[...], preferred_element_type=f32)
         + jnp.dot(h2, w1b_ref[...], preferred_element_type=f32)
         + jnp.dot(h3, w2a_ref[...], preferred_element_type=f32)
         + jnp.dot(h4, w2b_ref[...], preferred_element_type=f32)
         + bias_ref[...])
    gate = jnp.mean(jax.nn.sigmoid(z), axis=1, keepdims=True)
    ga = gate * attn_ref[...]
    mlo_ref[...] = sl * ga
    mhi_ref[...] = sh * ga


def _combine_body(agg_ref, cnt_ref, out_ref):
    a = jnp.concatenate([agg_ref[0], agg_ref[1]], axis=1)
    c = cnt_ref[0, :, 0:1]
    out_ref[...] = a / jnp.maximum(c, 1.0)


def kernel(x, edge_index, attn_value, ln_gamma, ln_beta, W, b):
    ei = edge_index.astype(jnp.int32)
    tgt = ei[0]
    src = ei[1]

    mesh = plsc.VectorSubcoreMesh(core_axis_name="c", subcore_axis_name="s")
    sc_params = pltpu.CompilerParams(use_tc_tiling_on_sc=False)

    gather = pl.kernel(
        _gather_body,
        out_type=[jax.ShapeDtypeStruct((E, D // 2), jnp.float32),
                  jax.ShapeDtypeStruct((E, D // 2), jnp.float32)],
        scratch_types=[pltpu.VMEM((2, EPW), jnp.int32),
                       pltpu.VMEM((2, GKG, CG, D // 2), jnp.float32),
                       pltpu.VMEM((2, GKG, CG, D // 2), jnp.float32),
                       pltpu.SemaphoreType.DMA,
                       pltpu.SemaphoreType.DMA,
                       pltpu.SemaphoreType.DMA],
        mesh=mesh,
        compiler_params=sc_params,
    )
    x16 = x.astype(jnp.bfloat16)
    xpair = jnp.stack([x16[:, :D // 2], x16[:, D // 2:]], axis=-1)
    xp = lax.bitcast_convert_type(xpair, jnp.float32)
    tf, sf = gather(xp, tgt, src)

    B = 2000
    HDm = D // 2
    g1a = ln_gamma[:HDm].reshape(1, HDm)
    g1b = ln_gamma[HDm:D].reshape(1, HDm)
    g2a = ln_gamma[D:D + HDm].reshape(1, HDm)
    g2b = ln_gamma[D + HDm:].reshape(1, HDm)
    b1a = ln_beta[:HDm].reshape(1, HDm)
    b1b = ln_beta[HDm:D].reshape(1, HDm)
    b2a = ln_beta[D:D + HDm].reshape(1, HDm)
    b2b = ln_beta[D + HDm:].reshape(1, HDm)
    w1a = W[:, :HDm].T
    w1b = W[:, HDm:D].T
    w2a = W[:, D:D + HDm].T
    w2b = W[:, D + HDm:].T
    bias = b.reshape(1, FD)
    attn2d = attn_value.reshape(E, 1)

    packed_spec = pl.BlockSpec((B, D // 2), lambda i: (i, 0))
    full = lambda shape: pl.BlockSpec(shape, lambda i: tuple(0 for _ in shape))
    msg_lo, msg_hi = pl.pallas_call(
        _dense_body,
        grid=(E // B,),
        in_specs=[packed_spec, packed_spec,
                  pl.BlockSpec((B, 1), lambda i: (i, 0))]
                 + [full((1, HDm))] * 8
                 + [full((HDm, FD))] * 4 + [full((1, FD))],
        out_specs=[packed_spec, packed_spec],
        out_shape=[jax.ShapeDtypeStruct((E, HD), jnp.float32),
                   jax.ShapeDtypeStruct((E, HD), jnp.float32)],
    )(tf, sf, attn2d, g1a, g1b, g2a, g2b, b1a, b1b, b2a, b2b,
      w1a, w1b, w2a, w2b, bias)

    zero_agg = jnp.zeros((NPT, HD), jnp.float32)
    zero_cnt = jnp.zeros((NPT, CW), jnp.float32)
    ones_rows = jnp.ones((CS, CW), jnp.float32)
    tgt3d = tgt.reshape(NS, EPT // CS, CS)
    scatter = pl.kernel(
        _scatter_body,
        out_type=[jax.ShapeDtypeStruct((NC, NP, HD), jnp.float32),
                  jax.ShapeDtypeStruct((NC, NP, CW), jnp.float32)],
        scratch_types=[pltpu.VMEM((EPT // CS, CS), jnp.int32),
                       pltpu.VMEM((2, GKS, CS, HD), jnp.float32),
                       pltpu.VMEM((CS, CW), jnp.float32),
                       pltpu.VMEM((NPT // 4, HD), jnp.float32),
                       pltpu.VMEM((NPT // 2, CW), jnp.float32),
                       pltpu.VMEM_SHARED((NP, HD), jnp.float32),
                       pltpu.VMEM_SHARED((NP, CW), jnp.float32),
                       pltpu.SemaphoreType.DMA,
                       pltpu.SemaphoreType.DMA,
                       pltpu.SemaphoreType.DMA],
        mesh=mesh,
        compiler_params=sc_params,
    )
    agg, cnt = scatter(msg_lo, msg_hi, tgt3d, zero_agg, zero_cnt, ones_rows)

    Bn = 2000
    out = pl.pallas_call(
        _combine_body,
        grid=(N // Bn,),
        in_specs=[pl.BlockSpec((NC, Bn, HD), lambda i: (0, i, 0)),
                  pl.BlockSpec((NC, Bn, CW), lambda i: (0, i, 0))],
        out_specs=pl.BlockSpec((Bn, D), lambda i: (i, 0)),
        out_shape=jax.ShapeDtypeStruct((N, D), jnp.float32),
    )(agg, cnt)
    return out


# speedup vs baseline: 1.7852x; 1.7852x over previous
import functools

import jax
import jax.numpy as jnp
from jax import lax
from jax.experimental import pallas as pl
from jax.experimental.pallas import tpu as pltpu
from jax.experimental.pallas import tpu_sc as plsc

N, E, D, FD = 10000, 320000, 128, 64
NC, NS = 2, 16
NW = NC * NS
H = E // 2
EPW = H // NW
NP = 10240
NPT = NP // NS
CW = 16
HD = D // 2
EPT = H // NS

CG = 40
GKG = 5
NGG = EPW // (CG * GKG)
CS = 40
GKS = 5
NGS = EPT // (CS * GKS)


def _gather_body(x_hbm, tgt_hbm, src_hbm, tf_hbm, sf_hbm,
                 idx_all, rows_t, rows_s, gsem, wsem0, wsem1):
    wid = lax.axis_index("s") * NC + lax.axis_index("c")
    base = wid * EPW
    pltpu.sync_copy(tgt_hbm.at[pl.ds(base, EPW)], idx_all.at[0])
    pltpu.sync_copy(src_hbm.at[pl.ds(base, EPW)], idx_all.at[1])

    def drain_writes(g, s):
        wsem = wsem0 if s == 0 else wsem1
        offp = base + g * GKG * CG
        for j in range(GKG):
            pltpu.make_async_copy(
                rows_t.at[s, j], tf_hbm.at[pl.ds(offp + j * CG, CG)],
                wsem).wait()
            pltpu.make_async_copy(
                rows_s.at[s, j], sf_hbm.at[pl.ds(offp + j * CG, CG)],
                wsem).wait()

    def phase(g, s):
        wsem = wsem0 if s == 0 else wsem1
        @pl.when(g >= 2)
        def _():
            drain_writes(g - 2, s)

        goff = g * GKG * CG
        handles = []
        for j in range(GKG):
            off = goff + j * CG
            handles.append(pltpu.async_copy(
                x_hbm.at[idx_all.at[0, pl.ds(off, CG)]], rows_t.at[s, j], gsem))
            handles.append(pltpu.async_copy(
                x_hbm.at[idx_all.at[1, pl.ds(off, CG)]], rows_s.at[s, j], gsem))
        for h in handles:
            h.wait()
        for j in range(GKG):
            off = base + goff + j * CG
            pltpu.async_copy(rows_t.at[s, j], tf_hbm.at[pl.ds(off, CG)], wsem)
            pltpu.async_copy(rows_s.at[s, j], sf_hbm.at[pl.ds(off, CG)], wsem)

    def body(h, carry):
        phase(2 * h, 0)
        phase(2 * h + 1, 1)
        return carry

    lax.fori_loop(0, NGG // 2, body, 0)
    if NGG % 2:
        phase(jnp.int32(NGG - 1), 0)
    drain_writes(jnp.int32(NGG - 2), (NGG - 2) % 2)
    drain_writes(jnp.int32(NGG - 1), (NGG - 1) % 2)


def _scatter_body(msg_hbm, tgt3d_hbm, zero_agg_hbm, zero_cnt_hbm, ones_hbm,
                  agg_hbm, cnt_hbm,
                  idx2d, rows_v, ones_v, wb_v, wbc_v, agg_sh, cnt_sh,
                  lsem, ssem0, ssem1):
    cid = lax.axis_index("c")
    sid = lax.axis_index("s")
    pltpu.sync_copy(zero_agg_hbm, agg_sh.at[pl.ds(sid * NPT, NPT)])
    pltpu.sync_copy(zero_cnt_hbm, cnt_sh.at[pl.ds(sid * NPT, NPT)])
    pltpu.sync_copy(ones_hbm, ones_v)
    pltpu.sync_copy(tgt3d_hbm.at[sid], idx2d)
    plsc.subcore_barrier()

    base = sid * EPT
    col0 = cid * HD

    def drain_scatters(s):
        ssem = ssem0 if s == 0 else ssem1
        for j in range(GKS):
            pltpu.make_async_copy(
                rows_v.at[s, j], agg_sh.at[pl.ds(0, CS)], ssem).wait()

            @pl.when(cid == 0)
            def _():
                pltpu.make_async_copy(
                    ones_v, cnt_sh.at[pl.ds(0, CS)], ssem).wait()

    def phase(g, s):
        @pl.when(g >= 2)
        def _():
            drain_scatters(s)

        goff = g * GKS * CS
        handles = []
        for j in range(GKS):
            off = base + goff + j * CS
            handles.append(pltpu.async_copy(
                msg_hbm.at[pl.ds(off, CS), pl.ds(col0, HD)],
                rows_v.at[s, j], lsem))
        for h in handles:
            h.wait()
        ssem = ssem0 if s == 0 else ssem1
        for j in range(GKS):
            pltpu.async_copy(
                rows_v.at[s, j], agg_sh.at[idx2d.at[g * GKS + j]], ssem,
                add=True)

            @pl.when(cid == 0)
            def _():
                pltpu.async_copy(
                    ones_v, cnt_sh.at[idx2d.at[g * GKS + j]], ssem,
                    add=True)

    def body(h, carry):
        phase(2 * h, 0)
        phase(2 * h + 1, 1)
        return carry

    lax.fori_loop(0, NGS // 2, body, 0)
    drain_scatters(0)
    drain_scatters(1)
    plsc.subcore_barrier()
    for k in range(4):
        q = NPT // 4
        r0 = sid * NPT + k * q
        pltpu.sync_copy(agg_sh.at[pl.ds(r0, q)], wb_v)
        pltpu.sync_copy(wb_v, agg_hbm.at[cid, pl.ds(r0, q)])
    for k in range(2):
        q = NPT // 2
        r0 = sid * NPT + k * q
        pltpu.sync_copy(cnt_sh.at[pl.ds(r0, q)], wbc_v)
        pltpu.sync_copy(wbc_v, cnt_hbm.at[cid, pl.ds(r0, q)])


def _dense_body(tf_ref, sf_ref, attn_ref, g1_ref, g2_ref, b1_ref, b2_ref,
                w1_ref, w2_ref, bias_ref, out_ref):
    tf = tf_ref[...]
    sf = sf_ref[...]
    s = jnp.sum(tf, axis=1, keepdims=True) + jnp.sum(sf, axis=1, keepdims=True)
    sq = (jnp.sum(tf * tf, axis=1, keepdims=True)
          + jnp.sum(sf * sf, axis=1, keepdims=True))
    mu = s * (1.0 / (2 * D))
    var = sq * (1.0 / (2 * D)) - mu * mu
    inv = lax.rsqrt(var + 1e-5)
    ht = jnp.maximum((tf - mu) * inv * g1_ref[...] + b1_ref[...], 0.0)
    hs = jnp.maximum((sf - mu) * inv * g2_ref[...] + b2_ref[...], 0.0)
    z = (jnp.dot(ht, w1_ref[...], preferred_element_type=jnp.float32)
         + jnp.dot(hs, w2_ref[...], preferred_element_type=jnp.float32)
         + bias_ref[...])
    gate = jnp.mean(jax.nn.sigmoid(z), axis=1, keepdims=True)
    out_ref[...] = sf * (gate * attn_ref[...])


def _combine_body(agg0_ref, agg1_ref, cnt0_ref, cnt1_ref, out_ref):
    a = jnp.concatenate([agg0_ref[0] + agg1_ref[0],
                         agg0_ref[1] + agg1_ref[1]], axis=1)
    c = cnt0_ref[0, :, 0:1] + cnt1_ref[0, :, 0:1]
    out_ref[...] = a / jnp.maximum(c, 1.0)


def kernel(x, edge_index, attn_value, ln_gamma, ln_beta, W, b):
    ei = edge_index.astype(jnp.int32)
    tgt = ei[0]
    src = ei[1]

    mesh = plsc.VectorSubcoreMesh(core_axis_name="c", subcore_axis_name="s")
    sc_params = pltpu.CompilerParams(use_tc_tiling_on_sc=False)

    gather = pl.kernel(
        _gather_body,
        out_type=[jax.ShapeDtypeStruct((H, D), jnp.float32),
                  jax.ShapeDtypeStruct((H, D), jnp.float32)],
        scratch_types=[pltpu.VMEM((2, EPW), jnp.int32),
                       pltpu.VMEM((2, GKG, CG, D), jnp.float32),
                       pltpu.VMEM((2, GKG, CG, D), jnp.float32),
                       pltpu.SemaphoreType.DMA,
                       pltpu.SemaphoreType.DMA,
                       pltpu.SemaphoreType.DMA],
        mesh=mesh,
        compiler_params=sc_params,
    )

    scatter = pl.kernel(
        _scatter_body,
        out_type=[jax.ShapeDtypeStruct((NC, NP, HD), jnp.float32),
                  jax.ShapeDtypeStruct((NC, NP, CW), jnp.float32)],
        scratch_types=[pltpu.VMEM((EPT // CS, CS), jnp.int32),
                       pltpu.VMEM((2, GKS, CS, HD), jnp.float32),
                       pltpu.VMEM((CS, CW), jnp.float32),
                       pltpu.VMEM((NPT // 4, HD), jnp.float32),
                       pltpu.VMEM((NPT // 2, CW), jnp.float32),
                       pltpu.VMEM_SHARED((NP, HD), jnp.float32),
                       pltpu.VMEM_SHARED((NP, CW), jnp.float32),
                       pltpu.SemaphoreType.DMA,
                       pltpu.SemaphoreType.DMA,
                       pltpu.SemaphoreType.DMA],
        mesh=mesh,
        compiler_params=sc_params,
    )

    B = 2000
    g1 = ln_gamma[:D].reshape(1, D)
    g2 = ln_gamma[D:].reshape(1, D)
    b1 = ln_beta[:D].reshape(1, D)
    b2 = ln_beta[D:].reshape(1, D)
    w1 = W[:, :D].T
    w2 = W[:, D:].T
    bias = b.reshape(1, FD)

    row_spec = pl.BlockSpec((B, D), lambda i: (i, 0))
    full = lambda shape: pl.BlockSpec(shape, lambda i: tuple(0 for _ in shape))
    dense = pl.pallas_call(
        _dense_body,
        grid=(H // B,),
        in_specs=[row_spec, row_spec,
                  pl.BlockSpec((B, 1), lambda i: (i, 0)),
                  full((1, D)), full((1, D)), full((1, D)), full((1, D)),
                  full((D, FD)), full((D, FD)), full((1, FD))],
        out_specs=row_spec,
        out_shape=jax.ShapeDtypeStruct((H, D), jnp.float32),
    )

    zero_agg = jnp.zeros((NPT, HD), jnp.float32)
    zero_cnt = jnp.zeros((NPT, CW), jnp.float32)
    ones_rows = jnp.ones((CS, CW), jnp.float32)

    parts = []
    for h in range(2):
        tgt_h = lax.slice_in_dim(tgt, h * H, (h + 1) * H)
        src_h = lax.slice_in_dim(src, h * H, (h + 1) * H)
        tf, sf = gather(x, tgt_h, src_h)
        attn2d = lax.slice_in_dim(attn_value, h * H, (h + 1) * H).reshape(H, 1)
        msg = dense(tf, sf, attn2d, g1, g2, b1, b2, w1, w2, bias)
        tgt3d = tgt_h.reshape(NS, EPT // CS, CS)
        agg, cnt = scatter(msg, tgt3d, zero_agg, zero_cnt, ones_rows)
        parts.append((agg, cnt))

    Bn = 2000
    out = pl.pallas_call(
        _combine_body,
        grid=(N // Bn,),
        in_specs=[pl.BlockSpec((NC, Bn, HD), lambda i: (0, i, 0)),
                  pl.BlockSpec((NC, Bn, HD), lambda i: (0, i, 0)),
                  pl.BlockSpec((NC, Bn, CW), lambda i: (0, i, 0)),
                  pl.BlockSpec((NC, Bn, CW), lambda i: (0, i, 0))],
        out_specs=pl.BlockSpec((Bn, D), lambda i: (i, 0)),
        out_shape=jax.ShapeDtypeStruct((N, D), jnp.float32),
    )(parts[0][0], parts[1][0], parts[0][1], parts[1][1])
    return out


# speedup vs baseline: 2.0025x; 1.1217x over previous
import functools

import jax
import jax.numpy as jnp
from jax import lax
from jax.experimental import pallas as pl
from jax.experimental.pallas import tpu as pltpu
from jax.experimental.pallas import tpu_sc as plsc

N, E, D, FD = 10000, 320000, 128, 64
NC, NS = 2, 16
NW = NC * NS
H = E // 2
HB = H // 2
EPW = H // NW
NP = 10240
NPT = NP // NS
CW = 16
HD = D // 2
EPT = H // NS

CG = 40
GKG = 5
NGG = EPW // (CG * GKG)
CS = 40
GKS = 5
NGS = EPT // (CS * GKS)


def _gather_body(x_hbm, tgt_hbm, src_hbm, tf_hbm, sf_hbm,
                 idx_all, rows_t, rows_s, gsem, wsem0, wsem1):
    wid = lax.axis_index("s") * NC + lax.axis_index("c")
    base = wid * EPW
    pltpu.sync_copy(tgt_hbm.at[pl.ds(base, EPW)], idx_all.at[0])
    pltpu.sync_copy(src_hbm.at[pl.ds(base, EPW)], idx_all.at[1])

    def drain_writes(g, s):
        wsem = wsem0 if s == 0 else wsem1
        offp = base + g * GKG * CG
        for j in range(GKG):
            pltpu.make_async_copy(
                rows_t.at[s, j], tf_hbm.at[pl.ds(offp + j * CG, CG)],
                wsem).wait()
            pltpu.make_async_copy(
                rows_s.at[s, j], sf_hbm.at[pl.ds(offp + j * CG, CG)],
                wsem).wait()

    def phase(g, s):
        wsem = wsem0 if s == 0 else wsem1
        @pl.when(g >= 2)
        def _():
            drain_writes(g - 2, s)

        goff = g * GKG * CG
        handles = []
        for j in range(GKG):
            off = goff + j * CG
            handles.append(pltpu.async_copy(
                x_hbm.at[idx_all.at[0, pl.ds(off, CG)]], rows_t.at[s, j], gsem))
            handles.append(pltpu.async_copy(
                x_hbm.at[idx_all.at[1, pl.ds(off, CG)]], rows_s.at[s, j], gsem))
        for h in handles:
            h.wait()
        for j in range(GKG):
            off = base + goff + j * CG
            pltpu.async_copy(rows_t.at[s, j], tf_hbm.at[pl.ds(off, CG)], wsem)
            pltpu.async_copy(rows_s.at[s, j], sf_hbm.at[pl.ds(off, CG)], wsem)

    def body(h, carry):
        phase(2 * h, 0)
        phase(2 * h + 1, 1)
        return carry

    lax.fori_loop(0, NGG // 2, body, 0)
    if NGG % 2:
        phase(jnp.int32(NGG - 1), 0)
    drain_writes(jnp.int32(NGG - 2), (NGG - 2) % 2)
    drain_writes(jnp.int32(NGG - 1), (NGG - 1) % 2)


def _scatter_body(mlo_hbm, mhi_hbm, tgt3d_hbm, zero_agg_hbm, zero_cnt_hbm,
                  ones_hbm, agg_hbm, cnt_hbm,
                  idx2d, rows_v, ones_v, wb_v, wbc_v, agg_sh, cnt_sh,
                  lsem, ssem0, ssem1):
    cid = lax.axis_index("c")
    sid = lax.axis_index("s")
    pltpu.sync_copy(zero_agg_hbm, agg_sh.at[pl.ds(sid * NPT, NPT)])
    pltpu.sync_copy(zero_cnt_hbm, cnt_sh.at[pl.ds(sid * NPT, NPT)])
    pltpu.sync_copy(ones_hbm, ones_v)
    pltpu.sync_copy(tgt3d_hbm.at[sid], idx2d)
    plsc.subcore_barrier()

    base = sid * EPT

    def drain_scatters(s):
        ssem = ssem0 if s == 0 else ssem1
        for j in range(GKS):
            pltpu.make_async_copy(
                rows_v.at[s, j], agg_sh.at[pl.ds(0, CS)], ssem).wait()

            @pl.when(cid == 0)
            def _():
                pltpu.make_async_copy(
                    ones_v, cnt_sh.at[pl.ds(0, CS)], ssem).wait()

    def phase(g, s):
        @pl.when(g >= 2)
        def _():
            drain_scatters(s)

        goff = g * GKS * CS

        @pl.when(cid == 0)
        def _():
            handles = []
            for j in range(GKS):
                off = base + goff + j * CS
                handles.append(pltpu.async_copy(
                    mlo_hbm.at[pl.ds(off, CS)], rows_v.at[s, j], lsem))
            for h in handles:
                h.wait()

        @pl.when(cid == 1)
        def _():
            handles = []
            for j in range(GKS):
                off = base + goff + j * CS
                handles.append(pltpu.async_copy(
                    mhi_hbm.at[pl.ds(off, CS)], rows_v.at[s, j], lsem))
            for h in handles:
                h.wait()

        ssem = ssem0 if s == 0 else ssem1
        for j in range(GKS):
            pltpu.async_copy(
                rows_v.at[s, j], agg_sh.at[idx2d.at[g * GKS + j]], ssem,
                add=True)

            @pl.when(cid == 0)
            def _():
                pltpu.async_copy(
                    ones_v, cnt_sh.at[idx2d.at[g * GKS + j]], ssem,
                    add=True)

    def body(h, carry):
        phase(2 * h, 0)
        phase(2 * h + 1, 1)
        return carry

    lax.fori_loop(0, NGS // 2, body, 0)
    drain_scatters(0)
    drain_scatters(1)
    plsc.subcore_barrier()
    for k in range(4):
        q = NPT // 4
        r0 = sid * NPT + k * q
        pltpu.sync_copy(agg_sh.at[pl.ds(r0, q)], wb_v)
        pltpu.sync_copy(wb_v, agg_hbm.at[cid, pl.ds(r0, q)])
    for k in range(2):
        q = NPT // 2
        r0 = sid * NPT + k * q
        pltpu.sync_copy(cnt_sh.at[pl.ds(r0, q)], wbc_v)
        pltpu.sync_copy(wbc_v, cnt_hbm.at[cid, pl.ds(r0, q)])


def _dense_body(tf_ref, sf_ref, attn_ref, o2_ref, o2t_ref,
                gl1_ref, gh1_ref, gl2_ref, gh2_ref,
                bl1_ref, bh1_ref, bl2_ref, bh2_ref, bias2_ref,
                bd1a_ref, bd1b_ref, bd2a_ref, bd2b_ref,
                mlo_ref, mhi_ref):
    bf = jnp.bfloat16
    f32 = jnp.float32

    def unpack(p):
        u = lax.bitcast_convert_type(p, jnp.uint32)
        lo = lax.bitcast_convert_type(u << 16, f32)
        hi = lax.bitcast_convert_type(u & jnp.uint32(0xFFFF0000), f32)
        return lo, hi

    tl, th = unpack(tf_ref[...])
    sl, sh = unpack(sf_ref[...])
    o2b = o2_ref[...].astype(bf)
    o2t = o2t_ref[...]
    ssum = tl + th + sl + sh
    sqs = tl * tl + th * th + sl * sl + sh * sh
    s2 = jnp.dot(ssum.astype(bf), o2b, preferred_element_type=f32)
    sq2 = jnp.dot(sqs.astype(bf), o2b, preferred_element_type=f32)
    mu2 = s2 * (1.0 / (2 * D))
    var2 = sq2 * (1.0 / (2 * D)) - mu2 * mu2
    inv2 = lax.rsqrt(var2 + 1e-5)
    mu = jnp.dot(mu2, o2t, preferred_element_type=f32)
    inv = jnp.dot(inv2, o2t, preferred_element_type=f32)
    h1 = jnp.maximum((tl - mu) * inv * gl1_ref[...] + bl1_ref[...], 0.0)
    h2 = jnp.maximum((th - mu) * inv * gh1_ref[...] + bh1_ref[...], 0.0)
    h3 = jnp.maximum((sl - mu) * inv * gl2_ref[...] + bl2_ref[...], 0.0)
    h4 = jnp.maximum((sh - mu) * inv * gh2_ref[...] + bh2_ref[...], 0.0)
    z = (jnp.dot(h1.astype(bf), bd1a_ref[...].astype(bf),
                 preferred_element_type=f32)
         + jnp.dot(h2.astype(bf), bd1b_ref[...].astype(bf),
                   preferred_element_type=f32)
         + jnp.dot(h3.astype(bf), bd2a_ref[...].astype(bf),
                   preferred_element_type=f32)
         + jnp.dot(h4.astype(bf), bd2b_ref[...].astype(bf),
                   preferred_element_type=f32)
         + bias2_ref[...])
    sig = jax.nn.sigmoid(z)
    gate2 = jnp.dot(sig.astype(bf), o2b, preferred_element_type=f32)
    ga2 = gate2 * (1.0 / FD) * attn_ref[...]
    ga = jnp.dot(ga2, o2t, preferred_element_type=f32)
    mlo_ref[...] = sl * ga
    mhi_ref[...] = sh * ga


def _combine_body(agg0_ref, agg1_ref, cnt0_ref, cnt1_ref, out_ref):
    a = jnp.concatenate([agg0_ref[0] + agg1_ref[0],
                         agg0_ref[1] + agg1_ref[1]], axis=1)
    c = cnt0_ref[0, :, 0:1] + cnt1_ref[0, :, 0:1]
    out_ref[...] = a / jnp.maximum(c, 1.0)


def kernel(x, edge_index, attn_value, ln_gamma, ln_beta, W, b):
    ei = edge_index.astype(jnp.int32)
    tgt = ei[0]
    src = ei[1]

    x16 = x.astype(jnp.bfloat16)
    xpk = lax.bitcast_convert_type(
        jnp.stack([x16[:, :HD], x16[:, HD:]], axis=-1), jnp.float32)

    mesh = plsc.VectorSubcoreMesh(core_axis_name="c", subcore_axis_name="s")
    sc_params = pltpu.CompilerParams(use_tc_tiling_on_sc=False)

    gather = pl.kernel(
        _gather_body,
        out_type=[jax.ShapeDtypeStruct((H, HD), jnp.float32),
                  jax.ShapeDtypeStruct((H, HD), jnp.float32)],
        scratch_types=[pltpu.VMEM((2, EPW), jnp.int32),
                       pltpu.VMEM((2, GKG, CG, HD), jnp.float32),
                       pltpu.VMEM((2, GKG, CG, HD), jnp.float32),
                       pltpu.SemaphoreType.DMA,
                       pltpu.SemaphoreType.DMA,
                       pltpu.SemaphoreType.DMA],
        mesh=mesh,
        compiler_params=sc_params,
    )

    scatter = pl.kernel(
        _scatter_body,
        out_type=[jax.ShapeDtypeStruct((NC, NP, HD), jnp.float32),
                  jax.ShapeDtypeStruct((NC, NP, CW), jnp.float32)],
        scratch_types=[pltpu.VMEM((EPT // CS, CS), jnp.int32),
                       pltpu.VMEM((2, GKS, CS, HD), jnp.float32),
                       pltpu.VMEM((CS, CW), jnp.float32),
                       pltpu.VMEM((NPT // 4, HD), jnp.float32),
                       pltpu.VMEM((NPT // 2, CW), jnp.float32),
                       pltpu.VMEM_SHARED((NP, HD), jnp.float32),
                       pltpu.VMEM_SHARED((NP, CW), jnp.float32),
                       pltpu.SemaphoreType.DMA,
                       pltpu.SemaphoreType.DMA,
                       pltpu.SemaphoreType.DMA],
        mesh=mesh,
        compiler_params=sc_params,
    )

    Bp = 1000
    o2 = jnp.concatenate(
        [jnp.concatenate([jnp.ones((HD, 1), jnp.float32),
                          jnp.zeros((HD, 1), jnp.float32)], axis=1),
         jnp.concatenate([jnp.zeros((HD, 1), jnp.float32),
                          jnp.ones((HD, 1), jnp.float32)], axis=1)], axis=0)
    o2t = o2.T

    def tile2(v):
        return jnp.concatenate([v, v]).reshape(1, D)

    gl1 = tile2(ln_gamma[:HD])
    gh1 = tile2(ln_gamma[HD:D])
    gl2 = tile2(ln_gamma[D:D + HD])
    gh2 = tile2(ln_gamma[D + HD:])
    bl1 = tile2(ln_beta[:HD])
    bh1 = tile2(ln_beta[HD:D])
    bl2 = tile2(ln_beta[D:D + HD])
    bh2 = tile2(ln_beta[D + HD:])
    bias2 = tile2(b)

    def blockdiag(a):
        za = jnp.zeros((HD, FD), jnp.float32)
        return jnp.concatenate(
            [jnp.concatenate([a, za], axis=1),
             jnp.concatenate([za, a], axis=1)], axis=0)

    bd1a = blockdiag(W[:, :HD].T)
    bd1b = blockdiag(W[:, HD:D].T)
    bd2a = blockdiag(W[:, D:D + HD].T)
    bd2b = blockdiag(W[:, D + HD:].T)

    row_spec = pl.BlockSpec((Bp, D), lambda i: (i, 0))
    full = lambda shape: pl.BlockSpec(shape, lambda i: tuple(0 for _ in shape))
    dense = pl.pallas_call(
        _dense_body,
        grid=(HB // Bp,),
        in_specs=[row_spec, row_spec,
                  pl.BlockSpec((Bp, 2), lambda i: (i, 0)),
                  full((D, 2)), full((2, D))]
                 + [full((1, D))] * 9
                 + [full((D, D))] * 4,
        out_specs=[row_spec, row_spec],
        out_shape=[jax.ShapeDtypeStruct((HB, D), jnp.float32),
                   jax.ShapeDtypeStruct((HB, D), jnp.float32)],
    )

    zero_agg = jnp.zeros((NPT, HD), jnp.float32)
    zero_cnt = jnp.zeros((NPT, CW), jnp.float32)
    ones_rows = jnp.ones((CS, CW), jnp.float32)

    parts = []
    for h in range(2):
        tgt_h = lax.slice_in_dim(tgt, h * H, (h + 1) * H)
        src_h = lax.slice_in_dim(src, h * H, (h + 1) * H)
        tf, sf = gather(xpk, tgt_h, src_h)
        tf2 = tf.reshape(HB, D)
        sf2 = sf.reshape(HB, D)
        attn2 = lax.slice_in_dim(attn_value, h * H, (h + 1) * H).reshape(HB, 2)
        mlo, mhi = dense(tf2, sf2, attn2, o2, o2t,
                         gl1, gh1, gl2, gh2, bl1, bh1, bl2, bh2, bias2,
                         bd1a, bd1b, bd2a, bd2b)
        tgt3d = tgt_h.reshape(NS, EPT // CS, CS)
        agg, cnt = scatter(mlo.reshape(H, HD), mhi.reshape(H, HD), tgt3d,
                           zero_agg, zero_cnt, ones_rows)
        parts.append((agg, cnt))

    Bn = 2000
    out = pl.pallas_call(
        _combine_body,
        grid=(N // Bn,),
        in_specs=[pl.BlockSpec((NC, Bn, HD), lambda i: (0, i, 0)),
                  pl.BlockSpec((NC, Bn, HD), lambda i: (0, i, 0)),
                  pl.BlockSpec((NC, Bn, CW), lambda i: (0, i, 0)),
                  pl.BlockSpec((NC, Bn, CW), lambda i: (0, i, 0))],
        out_specs=pl.BlockSpec((Bn, D), lambda i: (i, 0)),
        out_shape=jax.ShapeDtypeStruct((N, D), jnp.float32),
    )(parts[0][0], parts[1][0], parts[0][1], parts[1][1])
    return out


# speedup vs baseline: 2.1795x; 1.0884x over previous
import functools

import jax
import jax.numpy as jnp
from jax import lax
from jax.experimental import pallas as pl
from jax.experimental.pallas import tpu as pltpu
from jax.experimental.pallas import tpu_sc as plsc

N, E, D, FD = 10000, 320000, 128, 64
NC, NS = 2, 16
NW = NC * NS
H = E
HB = H // 2
EPW = H // NW
NP = 10240
NPT = NP // NS
CW = 16
HD = D // 2
EPT = H // NS

CG = 40
GKG = 5
NGG = EPW // (CG * GKG)
CS = 40
GKS = 5
NGS = EPT // (CS * GKS)


def _gather_body(x_hbm, tgt_hbm, src_hbm, tf_hbm, sf_hbm,
                 idx_all, rows_t, rows_s, gsem, wsem0, wsem1):
    wid = lax.axis_index("s") * NC + lax.axis_index("c")
    base = wid * EPW
    pltpu.sync_copy(tgt_hbm.at[pl.ds(base, EPW)], idx_all.at[0])
    pltpu.sync_copy(src_hbm.at[pl.ds(base, EPW)], idx_all.at[1])

    def drain_writes(g, s):
        wsem = wsem0 if s == 0 else wsem1
        offp = base + g * GKG * CG
        for j in range(GKG):
            pltpu.make_async_copy(
                rows_t.at[s, j], tf_hbm.at[pl.ds(offp + j * CG, CG)],
                wsem).wait()
            pltpu.make_async_copy(
                rows_s.at[s, j], sf_hbm.at[pl.ds(offp + j * CG, CG)],
                wsem).wait()

    def phase(g, s):
        wsem = wsem0 if s == 0 else wsem1
        @pl.when(g >= 2)
        def _():
            drain_writes(g - 2, s)

        goff = g * GKG * CG
        handles = []
        for j in range(GKG):
            off = goff + j * CG
            handles.append(pltpu.async_copy(
                x_hbm.at[idx_all.at[0, pl.ds(off, CG)]], rows_t.at[s, j], gsem))
            handles.append(pltpu.async_copy(
                x_hbm.at[idx_all.at[1, pl.ds(off, CG)]], rows_s.at[s, j], gsem))
        for h in handles:
            h.wait()
        for j in range(GKG):
            off = base + goff + j * CG
            pltpu.async_copy(rows_t.at[s, j], tf_hbm.at[pl.ds(off, CG)], wsem)
            pltpu.async_copy(rows_s.at[s, j], sf_hbm.at[pl.ds(off, CG)], wsem)

    def body(h, carry):
        phase(2 * h, 0)
        phase(2 * h + 1, 1)
        return carry

    lax.fori_loop(0, NGG // 2, body, 0)
    if NGG % 2:
        phase(jnp.int32(NGG - 1), 0)
    drain_writes(jnp.int32(NGG - 2), (NGG - 2) % 2)
    drain_writes(jnp.int32(NGG - 1), (NGG - 1) % 2)


def _scatter_body(mlo_hbm, mhi_hbm, tgt3d_hbm, zero_agg_hbm, zero_cnt_hbm,
                  ones_hbm, agg_hbm, cnt_hbm,
                  idx2d, rows_v, ones_v, wb_v, wbc_v, agg_sh, cnt_sh,
                  lsem, ssem0, ssem1):
    cid = lax.axis_index("c")
    sid = lax.axis_index("s")
    pltpu.sync_copy(zero_agg_hbm, agg_sh.at[pl.ds(sid * NPT, NPT)])
    pltpu.sync_copy(zero_cnt_hbm, cnt_sh.at[pl.ds(sid * NPT, NPT)])
    pltpu.sync_copy(ones_hbm, ones_v)
    pltpu.sync_copy(tgt3d_hbm.at[sid], idx2d)
    plsc.subcore_barrier()

    base = sid * EPT

    def drain_scatters(s):
        ssem = ssem0 if s == 0 else ssem1
        for j in range(GKS):
            pltpu.make_async_copy(
                rows_v.at[s, j], agg_sh.at[pl.ds(0, CS)], ssem).wait()

            @pl.when(cid == 0)
            def _():
                pltpu.make_async_copy(
                    ones_v, cnt_sh.at[pl.ds(0, CS)], ssem).wait()

    def phase(g, s):
        @pl.when(g >= 2)
        def _():
            drain_scatters(s)

        goff = g * GKS * CS

        @pl.when(cid == 0)
        def _():
            handles = []
            for j in range(GKS):
                off = base + goff + j * CS
                handles.append(pltpu.async_copy(
                    mlo_hbm.at[pl.ds(off, CS)], rows_v.at[s, j], lsem))
            for h in handles:
                h.wait()

        @pl.when(cid == 1)
        def _():
            handles = []
            for j in range(GKS):
                off = base + goff + j * CS
                handles.append(pltpu.async_copy(
                    mhi_hbm.at[pl.ds(off, CS)], rows_v.at[s, j], lsem))
            for h in handles:
                h.wait()

        ssem = ssem0 if s == 0 else ssem1
        for j in range(GKS):
            pltpu.async_copy(
                rows_v.at[s, j], agg_sh.at[idx2d.at[g * GKS + j]], ssem,
                add=True)

            @pl.when(cid == 0)
            def _():
                pltpu.async_copy(
                    ones_v, cnt_sh.at[idx2d.at[g * GKS + j]], ssem,
                    add=True)

    def body(h, carry):
        phase(2 * h, 0)
        phase(2 * h + 1, 1)
        return carry

    lax.fori_loop(0, NGS // 2, body, 0)
    drain_scatters(0)
    drain_scatters(1)
    plsc.subcore_barrier()
    for k in range(4):
        q = NPT // 4
        r0 = sid * NPT + k * q
        pltpu.sync_copy(agg_sh.at[pl.ds(r0, q)], wb_v)
        pltpu.sync_copy(wb_v, agg_hbm.at[cid, pl.ds(r0, q)])
    for k in range(2):
        q = NPT // 2
        r0 = sid * NPT + k * q
        pltpu.sync_copy(cnt_sh.at[pl.ds(r0, q)], wbc_v)
        pltpu.sync_copy(wbc_v, cnt_hbm.at[cid, pl.ds(r0, q)])


def _dense_body(tf_ref, sf_ref, attn_ref, o2_ref, o2t_ref,
                gl1_ref, gh1_ref, gl2_ref, gh2_ref,
                bl1_ref, bh1_ref, bl2_ref, bh2_ref, bias2_ref,
                bd1a_ref, bd1b_ref, bd2a_ref, bd2b_ref,
                mlo_ref, mhi_ref):
    bf = jnp.bfloat16
    f32 = jnp.float32

    def unpack(p):
        u = lax.bitcast_convert_type(p, jnp.uint32)
        lo = lax.bitcast_convert_type(u << 16, f32)
        hi = lax.bitcast_convert_type(u & jnp.uint32(0xFFFF0000), f32)
        return lo, hi

    tl, th = unpack(tf_ref[...])
    sl, sh = unpack(sf_ref[...])
    o2b = o2_ref[...].astype(bf)
    o2t = o2t_ref[...]
    ssum = tl + th + sl + sh
    sqs = tl * tl + th * th + sl * sl + sh * sh
    s2 = jnp.dot(ssum.astype(bf), o2b, preferred_element_type=f32)
    sq2 = jnp.dot(sqs.astype(bf), o2b, preferred_element_type=f32)
    mu2 = s2 * (1.0 / (2 * D))
    var2 = sq2 * (1.0 / (2 * D)) - mu2 * mu2
    inv2 = lax.rsqrt(var2 + 1e-5)
    mu = jnp.dot(mu2, o2t, preferred_element_type=f32)
    inv = jnp.dot(inv2, o2t, preferred_element_type=f32)
    h1 = jnp.maximum((tl - mu) * inv * gl1_ref[...] + bl1_ref[...], 0.0)
    h2 = jnp.maximum((th - mu) * inv * gh1_ref[...] + bh1_ref[...], 0.0)
    h3 = jnp.maximum((sl - mu) * inv * gl2_ref[...] + bl2_ref[...], 0.0)
    h4 = jnp.maximum((sh - mu) * inv * gh2_ref[...] + bh2_ref[...], 0.0)
    z = (jnp.dot(h1.astype(bf), bd1a_ref[...].astype(bf),
                 preferred_element_type=f32)
         + jnp.dot(h2.astype(bf), bd1b_ref[...].astype(bf),
                   preferred_element_type=f32)
         + jnp.dot(h3.astype(bf), bd2a_ref[...].astype(bf),
                   preferred_element_type=f32)
         + jnp.dot(h4.astype(bf), bd2b_ref[...].astype(bf),
                   preferred_element_type=f32)
         + bias2_ref[...])
    sig = jax.nn.sigmoid(z)
    gate2 = jnp.dot(sig.astype(bf), o2b, preferred_element_type=f32)
    ga2 = gate2 * (1.0 / FD) * attn_ref[...]
    ga = jnp.dot(ga2, o2t, preferred_element_type=f32)
    mlo_ref[...] = sl * ga
    mhi_ref[...] = sh * ga


def _combine_body(agg0_ref, cnt0_ref, out_ref):
    a = jnp.concatenate([agg0_ref[0], agg0_ref[1]], axis=1)
    c = cnt0_ref[0, :, 0:1]
    out_ref[...] = a / jnp.maximum(c, 1.0)


def kernel(x, edge_index, attn_value, ln_gamma, ln_beta, W, b):
    ei = edge_index.astype(jnp.int32)
    tgt = ei[0]
    src = ei[1]

    x16 = x.astype(jnp.bfloat16)
    xpk = lax.bitcast_convert_type(
        jnp.stack([x16[:, :HD], x16[:, HD:]], axis=-1), jnp.float32)

    mesh = plsc.VectorSubcoreMesh(core_axis_name="c", subcore_axis_name="s")
    sc_params = pltpu.CompilerParams(use_tc_tiling_on_sc=False)

    gather = pl.kernel(
        _gather_body,
        out_type=[jax.ShapeDtypeStruct((H, HD), jnp.float32),
                  jax.ShapeDtypeStruct((H, HD), jnp.float32)],
        scratch_types=[pltpu.VMEM((2, EPW), jnp.int32),
                       pltpu.VMEM((2, GKG, CG, HD), jnp.float32),
                       pltpu.VMEM((2, GKG, CG, HD), jnp.float32),
                       pltpu.SemaphoreType.DMA,
                       pltpu.SemaphoreType.DMA,
                       pltpu.SemaphoreType.DMA],
        mesh=mesh,
        compiler_params=sc_params,
    )

    scatter = pl.kernel(
        _scatter_body,
        out_type=[jax.ShapeDtypeStruct((NC, NP, HD), jnp.float32),
                  jax.ShapeDtypeStruct((NC, NP, CW), jnp.float32)],
        scratch_types=[pltpu.VMEM((EPT // CS, CS), jnp.int32),
                       pltpu.VMEM((2, GKS, CS, HD), jnp.float32),
                       pltpu.VMEM((CS, CW), jnp.float32),
                       pltpu.VMEM((NPT // 4, HD), jnp.float32),
                       pltpu.VMEM((NPT // 2, CW), jnp.float32),
                       pltpu.VMEM_SHARED((NP, HD), jnp.float32),
                       pltpu.VMEM_SHARED((NP, CW), jnp.float32),
                       pltpu.SemaphoreType.DMA,
                       pltpu.SemaphoreType.DMA,
                       pltpu.SemaphoreType.DMA],
        mesh=mesh,
        compiler_params=sc_params,
    )

    Bp = 1000
    o2 = jnp.concatenate(
        [jnp.concatenate([jnp.ones((HD, 1), jnp.float32),
                          jnp.zeros((HD, 1), jnp.float32)], axis=1),
         jnp.concatenate([jnp.zeros((HD, 1), jnp.float32),
                          jnp.ones((HD, 1), jnp.float32)], axis=1)], axis=0)
    o2t = o2.T

    def tile2(v):
        return jnp.concatenate([v, v]).reshape(1, D)

    gl1 = tile2(ln_gamma[:HD])
    gh1 = tile2(ln_gamma[HD:D])
    gl2 = tile2(ln_gamma[D:D + HD])
    gh2 = tile2(ln_gamma[D + HD:])
    bl1 = tile2(ln_beta[:HD])
    bh1 = tile2(ln_beta[HD:D])
    bl2 = tile2(ln_beta[D:D + HD])
    bh2 = tile2(ln_beta[D + HD:])
    bias2 = tile2(b)

    def blockdiag(a):
        za = jnp.zeros((HD, FD), jnp.float32)
        return jnp.concatenate(
            [jnp.concatenate([a, za], axis=1),
             jnp.concatenate([za, a], axis=1)], axis=0)

    bd1a = blockdiag(W[:, :HD].T)
    bd1b = blockdiag(W[:, HD:D].T)
    bd2a = blockdiag(W[:, D:D + HD].T)
    bd2b = blockdiag(W[:, D + HD:].T)

    row_spec = pl.BlockSpec((Bp, D), lambda i: (i, 0))
    full = lambda shape: pl.BlockSpec(shape, lambda i: tuple(0 for _ in shape))
    dense = pl.pallas_call(
        _dense_body,
        grid=(HB // Bp,),
        in_specs=[row_spec, row_spec,
                  pl.BlockSpec((Bp, 2), lambda i: (i, 0)),
                  full((D, 2)), full((2, D))]
                 + [full((1, D))] * 9
                 + [full((D, D))] * 4,
        out_specs=[row_spec, row_spec],
        out_shape=[jax.ShapeDtypeStruct((HB, D), jnp.float32),
                   jax.ShapeDtypeStruct((HB, D), jnp.float32)],
    )

    zero_agg = jnp.zeros((NPT, HD), jnp.float32)
    zero_cnt = jnp.zeros((NPT, CW), jnp.float32)
    ones_rows = jnp.ones((CS, CW), jnp.float32)

    parts = []
    for h in range(1):
        tgt_h = lax.slice_in_dim(tgt, h * H, (h + 1) * H)
        src_h = lax.slice_in_dim(src, h * H, (h + 1) * H)
        tf, sf = gather(xpk, tgt_h, src_h)
        tf2 = tf.reshape(HB, D)
        sf2 = sf.reshape(HB, D)
        attn2 = lax.slice_in_dim(attn_value, h * H, (h + 1) * H).reshape(HB, 2)
        mlo, mhi = dense(tf2, sf2, attn2, o2, o2t,
                         gl1, gh1, gl2, gh2, bl1, bh1, bl2, bh2, bias2,
                         bd1a, bd1b, bd2a, bd2b)
        tgt3d = tgt_h.reshape(NS, EPT // CS, CS)
        agg, cnt = scatter(mlo.reshape(H, HD), mhi.reshape(H, HD), tgt3d,
                           zero_agg, zero_cnt, ones_rows)
        parts.append((agg, cnt))

    Bn = 2000
    out = pl.pallas_call(
        _combine_body,
        grid=(N // Bn,),
        in_specs=[pl.BlockSpec((NC, Bn, HD), lambda i: (0, i, 0)),
                  pl.BlockSpec((NC, Bn, CW), lambda i: (0, i, 0))],
        out_specs=pl.BlockSpec((Bn, D), lambda i: (i, 0)),
        out_shape=jax.ShapeDtypeStruct((N, D), jnp.float32),
    )(parts[0][0], parts[0][1])
    return out


# speedup vs baseline: 2.3220x; 1.0654x over previous
import functools

import jax
import jax.numpy as jnp
from jax import lax
from jax.experimental import pallas as pl
from jax.experimental.pallas import tpu as pltpu
from jax.experimental.pallas import tpu_sc as plsc

N, E, D, FD = 10000, 320000, 128, 64
NC, NS = 2, 16
NW = NC * NS
H = E
HB = H // 2
EPW = H // NW
NP = 10240
NPT = NP // NS
CW = 16
HD = D // 2
EPT = H // NS

CG = 40
GKG = 5
NGG = EPW // (CG * GKG)
CS = 40
GKS = 5
NGS = EPT // (CS * GKS)


def _gather_body(x_hbm, tgt_hbm, src_hbm, tf_hbm, sf_hbm,
                 idx_all, rows_t, rows_s, gsem, wsem0, wsem1):
    wid = lax.axis_index("s") * NC + lax.axis_index("c")
    base = wid * EPW
    pltpu.sync_copy(tgt_hbm.at[pl.ds(base, EPW)], idx_all.at[0])
    pltpu.sync_copy(src_hbm.at[pl.ds(base, EPW)], idx_all.at[1])

    def drain_writes(g, s):
        wsem = wsem0 if s == 0 else wsem1
        offp = base + g * GKG * CG
        for j in range(GKG):
            pltpu.make_async_copy(
                rows_t.at[s, j], tf_hbm.at[pl.ds(offp + j * CG, CG)],
                wsem).wait()
            pltpu.make_async_copy(
                rows_s.at[s, j], sf_hbm.at[pl.ds(offp + j * CG, CG)],
                wsem).wait()

    def phase(g, s):
        wsem = wsem0 if s == 0 else wsem1
        @pl.when(g >= 2)
        def _():
            drain_writes(g - 2, s)

        goff = g * GKG * CG
        handles = []
        for j in range(GKG):
            off = goff + j * CG
            handles.append(pltpu.async_copy(
                x_hbm.at[idx_all.at[0, pl.ds(off, CG)]], rows_t.at[s, j], gsem))
            handles.append(pltpu.async_copy(
                x_hbm.at[idx_all.at[1, pl.ds(off, CG)]], rows_s.at[s, j], gsem))
        for h in handles:
            h.wait()
        for j in range(GKG):
            off = base + goff + j * CG
            pltpu.async_copy(rows_t.at[s, j], tf_hbm.at[pl.ds(off, CG)], wsem)
            pltpu.async_copy(rows_s.at[s, j], sf_hbm.at[pl.ds(off, CG)], wsem)

    def body(h, carry):
        phase(2 * h, 0)
        phase(2 * h + 1, 1)
        return carry

    lax.fori_loop(0, NGG // 2, body, 0)
    if NGG % 2:
        phase(jnp.int32(NGG - 1), 0)
    drain_writes(jnp.int32(NGG - 2), (NGG - 2) % 2)
    drain_writes(jnp.int32(NGG - 1), (NGG - 1) % 2)


def _scatter_body(mlo_hbm, mhi_hbm, tgt3d_hbm, zero_agg_hbm, zero_cnt_hbm,
                  ones_hbm, agg_hbm, cnt_hbm,
                  idx2d, rows_v, ones_v, wb_v, wbc_v, agg_sh, cnt_sh,
                  lsem, ssem0, ssem1):
    cid = lax.axis_index("c")
    sid = lax.axis_index("s")
    pltpu.sync_copy(zero_agg_hbm, agg_sh.at[pl.ds(sid * NPT, NPT)])
    pltpu.sync_copy(zero_cnt_hbm, cnt_sh.at[pl.ds(sid * NPT, NPT)])
    pltpu.sync_copy(ones_hbm, ones_v)
    pltpu.sync_copy(tgt3d_hbm.at[sid], idx2d)
    plsc.subcore_barrier()

    base = sid * EPT

    def drain_scatters(s):
        ssem = ssem0 if s == 0 else ssem1
        for j in range(GKS):
            pltpu.make_async_copy(
                rows_v.at[s, j], agg_sh.at[pl.ds(0, CS)], ssem).wait()

            @pl.when(cid == 0)
            def _():
                pltpu.make_async_copy(
                    ones_v, cnt_sh.at[pl.ds(0, CS)], ssem).wait()

    def phase(g, s):
        @pl.when(g >= 2)
        def _():
            drain_scatters(s)

        goff = g * GKS * CS

        @pl.when(cid == 0)
        def _():
            handles = []
            for j in range(GKS):
                off = base + goff + j * CS
                handles.append(pltpu.async_copy(
                    mlo_hbm.at[pl.ds(off, CS)], rows_v.at[s, j], lsem))
            for h in handles:
                h.wait()

        @pl.when(cid == 1)
        def _():
            handles = []
            for j in range(GKS):
                off = base + goff + j * CS
                handles.append(pltpu.async_copy(
                    mhi_hbm.at[pl.ds(off, CS)], rows_v.at[s, j], lsem))
            for h in handles:
                h.wait()

        ssem = ssem0 if s == 0 else ssem1
        for j in range(GKS):
            pltpu.async_copy(
                rows_v.at[s, j], agg_sh.at[idx2d.at[g * GKS + j]], ssem,
                add=True)

            @pl.when(cid == 0)
            def _():
                pltpu.async_copy(
                    ones_v, cnt_sh.at[idx2d.at[g * GKS + j]], ssem,
                    add=True)

    def body(h, carry):
        phase(2 * h, 0)
        phase(2 * h + 1, 1)
        return carry

    lax.fori_loop(0, NGS // 2, body, 0)
    drain_scatters(0)
    drain_scatters(1)
    plsc.subcore_barrier()
    for k in range(4):
        q = NPT // 4
        r0 = sid * NPT + k * q
        pltpu.sync_copy(agg_sh.at[pl.ds(r0, q)], wb_v)
        pltpu.sync_copy(wb_v, agg_hbm.at[cid, pl.ds(r0, q)])
    for k in range(2):
        q = NPT // 2
        r0 = sid * NPT + k * q
        pltpu.sync_copy(cnt_sh.at[pl.ds(r0, q)], wbc_v)
        pltpu.sync_copy(wbc_v, cnt_hbm.at[cid, pl.ds(r0, q)])


def _dense_body(tf_ref, sf_ref, attn_ref, o2_ref, o2t_ref,
                gl1_ref, gh1_ref, gl2_ref, gh2_ref,
                bl1_ref, bh1_ref, bl2_ref, bh2_ref, bias2_ref,
                bd1a_ref, bd1b_ref, bd2a_ref, bd2b_ref,
                mlo_ref, mhi_ref):
    bf = jnp.bfloat16
    f32 = jnp.float32

    def unpack(p):
        u = lax.bitcast_convert_type(p, jnp.uint32)
        lo = lax.bitcast_convert_type(u << 16, f32)
        hi = lax.bitcast_convert_type(u & jnp.uint32(0xFFFF0000), f32)
        return lo, hi

    tl, th = unpack(tf_ref[...])
    sl, sh = unpack(sf_ref[...])
    o2b = o2_ref[...].astype(bf)
    o2t = o2t_ref[...]
    ssum = tl + th + sl + sh
    sqs = tl * tl + th * th + sl * sl + sh * sh
    s2 = jnp.dot(ssum.astype(bf), o2b, preferred_element_type=f32)
    sq2 = jnp.dot(sqs.astype(bf), o2b, preferred_element_type=f32)
    mu2 = s2 * (1.0 / (2 * D))
    var2 = sq2 * (1.0 / (2 * D)) - mu2 * mu2
    inv2 = lax.rsqrt(var2 + 1e-5)
    mu = jnp.dot(mu2, o2t, preferred_element_type=f32)
    inv = jnp.dot(inv2, o2t, preferred_element_type=f32)
    h1 = jnp.maximum((tl - mu) * inv * gl1_ref[...] + bl1_ref[...], 0.0)
    h2 = jnp.maximum((th - mu) * inv * gh1_ref[...] + bh1_ref[...], 0.0)
    h3 = jnp.maximum((sl - mu) * inv * gl2_ref[...] + bl2_ref[...], 0.0)
    h4 = jnp.maximum((sh - mu) * inv * gh2_ref[...] + bh2_ref[...], 0.0)
    z = (jnp.dot(h1.astype(bf), bd1a_ref[...].astype(bf),
                 preferred_element_type=f32)
         + jnp.dot(h2.astype(bf), bd1b_ref[...].astype(bf),
                   preferred_element_type=f32)
         + jnp.dot(h3.astype(bf), bd2a_ref[...].astype(bf),
                   preferred_element_type=f32)
         + jnp.dot(h4.astype(bf), bd2b_ref[...].astype(bf),
                   preferred_element_type=f32)
         + bias2_ref[...])
    sig = jax.nn.sigmoid(z)
    gate2 = jnp.dot(sig.astype(bf), o2b, preferred_element_type=f32)
    ga2 = gate2 * (1.0 / FD) * attn_ref[...]
    ga = jnp.dot(ga2, o2t, preferred_element_type=f32)
    mlo_ref[...] = sl * ga
    mhi_ref[...] = sh * ga


def _combine_body(agg0_ref, cnt0_ref, out_ref):
    a = jnp.concatenate([agg0_ref[0], agg0_ref[1]], axis=1)
    c = cnt0_ref[0, :, 0:1]
    out_ref[...] = a / jnp.maximum(c, 1.0)


def kernel(x, edge_index, attn_value, ln_gamma, ln_beta, W, b):
    ei = edge_index.astype(jnp.int32)
    tgt = ei[0]
    src = ei[1]

    x16 = x.astype(jnp.bfloat16)
    xpk = lax.bitcast_convert_type(
        jnp.stack([x16[:, :HD], x16[:, HD:]], axis=-1), jnp.float32)

    mesh = plsc.VectorSubcoreMesh(core_axis_name="c", subcore_axis_name="s")
    sc_params = pltpu.CompilerParams(use_tc_tiling_on_sc=False)

    gather = pl.kernel(
        _gather_body,
        out_type=[jax.ShapeDtypeStruct((H, HD), jnp.float32),
                  jax.ShapeDtypeStruct((H, HD), jnp.float32)],
        scratch_types=[pltpu.VMEM((2, EPW), jnp.int32),
                       pltpu.VMEM((2, GKG, CG, HD), jnp.float32),
                       pltpu.VMEM((2, GKG, CG, HD), jnp.float32),
                       pltpu.SemaphoreType.DMA,
                       pltpu.SemaphoreType.DMA,
                       pltpu.SemaphoreType.DMA],
        mesh=mesh,
        compiler_params=sc_params,
    )

    scatter = pl.kernel(
        _scatter_body,
        out_type=[jax.ShapeDtypeStruct((NC, NP, HD), jnp.float32),
                  jax.ShapeDtypeStruct((NC, NP, CW), jnp.float32)],
        scratch_types=[pltpu.VMEM((EPT // CS, CS), jnp.int32),
                       pltpu.VMEM((2, GKS, CS, HD), jnp.float32),
                       pltpu.VMEM((CS, CW), jnp.float32),
                       pltpu.VMEM((NPT // 4, HD), jnp.float32),
                       pltpu.VMEM((NPT // 2, CW), jnp.float32),
                       pltpu.VMEM_SHARED((NP, HD), jnp.float32),
                       pltpu.VMEM_SHARED((NP, CW), jnp.float32),
                       pltpu.SemaphoreType.DMA,
                       pltpu.SemaphoreType.DMA,
                       pltpu.SemaphoreType.DMA],
        mesh=mesh,
        compiler_params=sc_params,
    )

    Bp = 2000
    o2 = jnp.concatenate(
        [jnp.concatenate([jnp.ones((HD, 1), jnp.float32),
                          jnp.zeros((HD, 1), jnp.float32)], axis=1),
         jnp.concatenate([jnp.zeros((HD, 1), jnp.float32),
                          jnp.ones((HD, 1), jnp.float32)], axis=1)], axis=0)
    o2t = o2.T

    def tile2(v):
        return jnp.concatenate([v, v]).reshape(1, D)

    gl1 = tile2(ln_gamma[:HD])
    gh1 = tile2(ln_gamma[HD:D])
    gl2 = tile2(ln_gamma[D:D + HD])
    gh2 = tile2(ln_gamma[D + HD:])
    bl1 = tile2(ln_beta[:HD])
    bh1 = tile2(ln_beta[HD:D])
    bl2 = tile2(ln_beta[D:D + HD])
    bh2 = tile2(ln_beta[D + HD:])
    bias2 = tile2(b)

    def blockdiag(a):
        za = jnp.zeros((HD, FD), jnp.float32)
        return jnp.concatenate(
            [jnp.concatenate([a, za], axis=1),
             jnp.concatenate([za, a], axis=1)], axis=0)

    bd1a = blockdiag(W[:, :HD].T)
    bd1b = blockdiag(W[:, HD:D].T)
    bd2a = blockdiag(W[:, D:D + HD].T)
    bd2b = blockdiag(W[:, D + HD:].T)

    row_spec = pl.BlockSpec((Bp, D), lambda i: (i, 0))
    full = lambda shape: pl.BlockSpec(shape, lambda i: tuple(0 for _ in shape))
    dense = pl.pallas_call(
        _dense_body,
        grid=(HB // Bp,),
        in_specs=[row_spec, row_spec,
                  pl.BlockSpec((Bp, 2), lambda i: (i, 0)),
                  full((D, 2)), full((2, D))]
                 + [full((1, D))] * 9
                 + [full((D, D))] * 4,
        out_specs=[row_spec, row_spec],
        out_shape=[jax.ShapeDtypeStruct((HB, D), jnp.float32),
                   jax.ShapeDtypeStruct((HB, D), jnp.float32)],
    )

    zero_agg = jnp.zeros((NPT, HD), jnp.float32)
    zero_cnt = jnp.zeros((NPT, CW), jnp.float32)
    ones_rows = jnp.ones((CS, CW), jnp.float32)

    parts = []
    for h in range(1):
        tgt_h = lax.slice_in_dim(tgt, h * H, (h + 1) * H)
        src_h = lax.slice_in_dim(src, h * H, (h + 1) * H)
        tf, sf = gather(xpk, tgt_h, src_h)
        tf2 = tf.reshape(HB, D)
        sf2 = sf.reshape(HB, D)
        attn2 = lax.slice_in_dim(attn_value, h * H, (h + 1) * H).reshape(HB, 2)
        mlo, mhi = dense(tf2, sf2, attn2, o2, o2t,
                         gl1, gh1, gl2, gh2, bl1, bh1, bl2, bh2, bias2,
                         bd1a, bd1b, bd2a, bd2b)
        tgt3d = tgt_h.reshape(NS, EPT // CS, CS)
        agg, cnt = scatter(mlo.reshape(H, HD), mhi.reshape(H, HD), tgt3d,
                           zero_agg, zero_cnt, ones_rows)
        parts.append((agg, cnt))

    Bn = 2000
    out = pl.pallas_call(
        _combine_body,
        grid=(N // Bn,),
        in_specs=[pl.BlockSpec((NC, Bn, HD), lambda i: (0, i, 0)),
                  pl.BlockSpec((NC, Bn, CW), lambda i: (0, i, 0))],
        out_specs=pl.BlockSpec((Bn, D), lambda i: (i, 0)),
        out_shape=jax.ShapeDtypeStruct((N, D), jnp.float32),
    )(parts[0][0], parts[0][1])
    return out


# speedup vs baseline: 2.4870x; 1.0711x over previous
import functools

import jax
import jax.numpy as jnp
from jax import lax
from jax.experimental import pallas as pl
from jax.experimental.pallas import tpu as pltpu
from jax.experimental.pallas import tpu_sc as plsc

N, E, D, FD = 10000, 320000, 128, 64
NC, NS = 2, 16
NW = NC * NS
H = E
HB = H // 2
EPW = H // NW
NP = 10240
NPT = NP // NS
CW = 16
HD = D // 2
EPT = H // NS

CG = 40
GKG = 5
NGG = EPW // (CG * GKG)
CS = 40
GKS = 5
NGS = EPT // (CS * GKS)


def _gather_body(x_hbm, tgt_hbm, src_hbm, tf_hbm, sf_hbm,
                 idx_all, rows_t, rows_s, gsem, wsem0, wsem1):
    wid = lax.axis_index("s") * NC + lax.axis_index("c")
    base = wid * EPW
    pltpu.sync_copy(tgt_hbm.at[pl.ds(base, EPW)], idx_all.at[0])
    pltpu.sync_copy(src_hbm.at[pl.ds(base, EPW)], idx_all.at[1])

    def drain_writes(g, s):
        wsem = wsem0 if s == 0 else wsem1
        offp = base + g * GKG * CG
        for j in range(GKG):
            pltpu.make_async_copy(
                rows_t.at[s, j], tf_hbm.at[pl.ds(offp + j * CG, CG)],
                wsem).wait()
            pltpu.make_async_copy(
                rows_s.at[s, j], sf_hbm.at[pl.ds(offp + j * CG, CG)],
                wsem).wait()

    def phase(g, s):
        wsem = wsem0 if s == 0 else wsem1
        @pl.when(g >= 2)
        def _():
            drain_writes(g - 2, s)

        goff = g * GKG * CG
        handles = []
        for j in range(GKG):
            off = goff + j * CG
            handles.append(pltpu.async_copy(
                x_hbm.at[idx_all.at[0, pl.ds(off, CG)]], rows_t.at[s, j], gsem))
            handles.append(pltpu.async_copy(
                x_hbm.at[idx_all.at[1, pl.ds(off, CG)]], rows_s.at[s, j], gsem))
        for h in handles:
            h.wait()
        for j in range(GKG):
            off = base + goff + j * CG
            pltpu.async_copy(rows_t.at[s, j], tf_hbm.at[pl.ds(off, CG)], wsem)
            pltpu.async_copy(rows_s.at[s, j], sf_hbm.at[pl.ds(off, CG)], wsem)

    def body(h, carry):
        phase(2 * h, 0)
        phase(2 * h + 1, 1)
        return carry

    lax.fori_loop(0, NGG // 2, body, 0)
    if NGG % 2:
        phase(jnp.int32(NGG - 1), 0)
    drain_writes(jnp.int32(NGG - 2), (NGG - 2) % 2)
    drain_writes(jnp.int32(NGG - 1), (NGG - 1) % 2)


def _scatter_body(mlo_hbm, mhi_hbm, tgt3d_hbm, zero_agg_hbm, zero_cnt_hbm,
                  ones_hbm, agg_hbm, cnt_hbm,
                  idx2d, rows_v, ones_v, wb_v, wbc_v, agg_sh, cnt_sh,
                  lsem, ssem0, ssem1):
    cid = lax.axis_index("c")
    sid = lax.axis_index("s")
    pltpu.sync_copy(zero_agg_hbm, agg_sh.at[pl.ds(sid * NPT, NPT)])
    pltpu.sync_copy(zero_cnt_hbm, cnt_sh.at[pl.ds(sid * NPT, NPT)])
    pltpu.sync_copy(ones_hbm, ones_v)
    pltpu.sync_copy(tgt3d_hbm.at[sid], idx2d)
    plsc.subcore_barrier()

    base = sid * EPT

    def drain_scatters(s):
        ssem = ssem0 if s == 0 else ssem1
        for j in range(GKS):
            pltpu.make_async_copy(
                rows_v.at[s, j], agg_sh.at[pl.ds(0, CS)], ssem).wait()

            @pl.when(cid == 0)
            def _():
                pltpu.make_async_copy(
                    ones_v, cnt_sh.at[pl.ds(0, CS)], ssem).wait()

    def phase(g, s):
        @pl.when(g >= 2)
        def _():
            drain_scatters(s)

        goff = g * GKS * CS

        @pl.when(cid == 0)
        def _():
            handles = []
            for j in range(GKS):
                off = base + goff + j * CS
                handles.append(pltpu.async_copy(
                    mlo_hbm.at[pl.ds(off, CS)], rows_v.at[s, j], lsem))
            for h in handles:
                h.wait()

        @pl.when(cid == 1)
        def _():
            handles = []
            for j in range(GKS):
                off = base + goff + j * CS
                handles.append(pltpu.async_copy(
                    mhi_hbm.at[pl.ds(off, CS)], rows_v.at[s, j], lsem))
            for h in handles:
                h.wait()

        ssem = ssem0 if s == 0 else ssem1
        for j in range(GKS):
            pltpu.async_copy(
                rows_v.at[s, j], agg_sh.at[idx2d.at[g * GKS + j]], ssem,
                add=True)

            @pl.when(cid == 0)
            def _():
                pltpu.async_copy(
                    ones_v, cnt_sh.at[idx2d.at[g * GKS + j]], ssem,
                    add=True)

    def body(h, carry):
        phase(2 * h, 0)
        phase(2 * h + 1, 1)
        return carry

    lax.fori_loop(0, NGS // 2, body, 0)
    drain_scatters(0)
    drain_scatters(1)
    plsc.subcore_barrier()
    for k in range(4):
        q = NPT // 4
        r0 = sid * NPT + k * q
        pltpu.sync_copy(agg_sh.at[pl.ds(r0, q)], wb_v)
        pltpu.sync_copy(wb_v, agg_hbm.at[cid, pl.ds(r0, q)])
    for k in range(2):
        q = NPT // 2
        r0 = sid * NPT + k * q
        pltpu.sync_copy(cnt_sh.at[pl.ds(r0, q)], wbc_v)
        pltpu.sync_copy(wbc_v, cnt_hbm.at[cid, pl.ds(r0, q)])


def _dense_body(tf_ref, sf_ref, attn_ref, o2_ref, o2t_ref,
                gl1_ref, gh1_ref, gl2_ref, gh2_ref,
                bl1_ref, bh1_ref, bl2_ref, bh2_ref, bias2_ref,
                bd1a_ref, bd1b_ref, bd2a_ref, bd2b_ref,
                mlo_ref, mhi_ref):
    bf = jnp.bfloat16
    f32 = jnp.float32

    def unpack(p):
        u = lax.bitcast_convert_type(p, jnp.uint32)
        lo = lax.bitcast_convert_type(u << 16, f32)
        hi = lax.bitcast_convert_type(u & jnp.uint32(0xFFFF0000), f32)
        return lo, hi

    tl, th = unpack(tf_ref[...])
    sl, sh = unpack(sf_ref[...])
    o2b = o2_ref[...].astype(bf)
    o2t = o2t_ref[...]
    ssum = tl + th + sl + sh
    sqs = tl * tl + th * th + sl * sl + sh * sh
    s2 = jnp.dot(ssum.astype(bf), o2b, preferred_element_type=f32)
    sq2 = jnp.dot(sqs.astype(bf), o2b, preferred_element_type=f32)
    mu2 = s2 * (1.0 / (2 * D))
    var2 = sq2 * (1.0 / (2 * D)) - mu2 * mu2
    inv2 = lax.rsqrt(var2 + 1e-5)
    mu = jnp.dot(mu2, o2t, preferred_element_type=f32)
    inv = jnp.dot(inv2, o2t, preferred_element_type=f32)
    h1 = jnp.maximum((tl - mu) * inv * gl1_ref[...] + bl1_ref[...], 0.0)
    h2 = jnp.maximum((th - mu) * inv * gh1_ref[...] + bh1_ref[...], 0.0)
    h3 = jnp.maximum((sl - mu) * inv * gl2_ref[...] + bl2_ref[...], 0.0)
    h4 = jnp.maximum((sh - mu) * inv * gh2_ref[...] + bh2_ref[...], 0.0)
    z = (jnp.dot(h1.astype(bf), bd1a_ref[...].astype(bf),
                 preferred_element_type=f32)
         + jnp.dot(h2.astype(bf), bd1b_ref[...].astype(bf),
                   preferred_element_type=f32)
         + jnp.dot(h3.astype(bf), bd2a_ref[...].astype(bf),
                   preferred_element_type=f32)
         + jnp.dot(h4.astype(bf), bd2b_ref[...].astype(bf),
                   preferred_element_type=f32)
         + bias2_ref[...])
    sig = jax.nn.sigmoid(z)
    gate2 = jnp.dot(sig.astype(bf), o2b, preferred_element_type=f32)
    ga2 = gate2 * (1.0 / FD) * attn_ref[...]
    ga = jnp.dot(ga2, o2t, preferred_element_type=f32)
    mlo_ref[...] = sl * ga
    mhi_ref[...] = sh * ga


def _combine_body(agg0_ref, cnt0_ref, out_ref):
    a = jnp.concatenate([agg0_ref[0], agg0_ref[1]], axis=1)
    c = cnt0_ref[0, :, 0:1]
    out_ref[...] = a / jnp.maximum(c, 1.0)


def kernel(x, edge_index, attn_value, ln_gamma, ln_beta, W, b):
    ei = edge_index.astype(jnp.int32)
    tgt = ei[0]
    src = ei[1]

    x16 = x.astype(jnp.bfloat16)
    xpk = lax.bitcast_convert_type(
        jnp.stack([x16[:, :HD], x16[:, HD:]], axis=-1), jnp.float32)

    mesh = plsc.VectorSubcoreMesh(core_axis_name="c", subcore_axis_name="s")
    sc_params = pltpu.CompilerParams(use_tc_tiling_on_sc=False)

    gather = pl.kernel(
        _gather_body,
        out_type=[jax.ShapeDtypeStruct((H, HD), jnp.float32),
                  jax.ShapeDtypeStruct((H, HD), jnp.float32)],
        scratch_types=[pltpu.VMEM((2, EPW), jnp.int32),
                       pltpu.VMEM((2, GKG, CG, HD), jnp.float32),
                       pltpu.VMEM((2, GKG, CG, HD), jnp.float32),
                       pltpu.SemaphoreType.DMA,
                       pltpu.SemaphoreType.DMA,
                       pltpu.SemaphoreType.DMA],
        mesh=mesh,
        compiler_params=sc_params,
    )

    scatter = pl.kernel(
        _scatter_body,
        out_type=[jax.ShapeDtypeStruct((NC, NP, HD), jnp.float32),
                  jax.ShapeDtypeStruct((NC, NP, CW), jnp.float32)],
        scratch_types=[pltpu.VMEM((EPT // CS, CS), jnp.int32),
                       pltpu.VMEM((2, GKS, CS, HD), jnp.float32),
                       pltpu.VMEM((CS, CW), jnp.float32),
                       pltpu.VMEM((NPT // 4, HD), jnp.float32),
                       pltpu.VMEM((NPT // 2, CW), jnp.float32),
                       pltpu.VMEM_SHARED((NP, HD), jnp.float32),
                       pltpu.VMEM_SHARED((NP, CW), jnp.float32),
                       pltpu.SemaphoreType.DMA,
                       pltpu.SemaphoreType.DMA,
                       pltpu.SemaphoreType.DMA],
        mesh=mesh,
        compiler_params=sc_params,
    )

    Bp = 4000
    o2 = jnp.concatenate(
        [jnp.concatenate([jnp.ones((HD, 1), jnp.float32),
                          jnp.zeros((HD, 1), jnp.float32)], axis=1),
         jnp.concatenate([jnp.zeros((HD, 1), jnp.float32),
                          jnp.ones((HD, 1), jnp.float32)], axis=1)], axis=0)
    o2t = o2.T

    def tile2(v):
        return jnp.concatenate([v, v]).reshape(1, D)

    gl1 = tile2(ln_gamma[:HD])
    gh1 = tile2(ln_gamma[HD:D])
    gl2 = tile2(ln_gamma[D:D + HD])
    gh2 = tile2(ln_gamma[D + HD:])
    bl1 = tile2(ln_beta[:HD])
    bh1 = tile2(ln_beta[HD:D])
    bl2 = tile2(ln_beta[D:D + HD])
    bh2 = tile2(ln_beta[D + HD:])
    bias2 = tile2(b)

    def blockdiag(a):
        za = jnp.zeros((HD, FD), jnp.float32)
        return jnp.concatenate(
            [jnp.concatenate([a, za], axis=1),
             jnp.concatenate([za, a], axis=1)], axis=0)

    bd1a = blockdiag(W[:, :HD].T)
    bd1b = blockdiag(W[:, HD:D].T)
    bd2a = blockdiag(W[:, D:D + HD].T)
    bd2b = blockdiag(W[:, D + HD:].T)

    row_spec = pl.BlockSpec((Bp, D), lambda i: (i, 0))
    full = lambda shape: pl.BlockSpec(shape, lambda i: tuple(0 for _ in shape))
    dense = pl.pallas_call(
        _dense_body,
        grid=(HB // Bp,),
        in_specs=[row_spec, row_spec,
                  pl.BlockSpec((Bp, 2), lambda i: (i, 0)),
                  full((D, 2)), full((2, D))]
                 + [full((1, D))] * 9
                 + [full((D, D))] * 4,
        out_specs=[row_spec, row_spec],
        out_shape=[jax.ShapeDtypeStruct((HB, D), jnp.float32),
                   jax.ShapeDtypeStruct((HB, D), jnp.float32)],
    )

    zero_agg = jnp.zeros((NPT, HD), jnp.float32)
    zero_cnt = jnp.zeros((NPT, CW), jnp.float32)
    ones_rows = jnp.ones((CS, CW), jnp.float32)

    parts = []
    for h in range(1):
        tgt_h = lax.slice_in_dim(tgt, h * H, (h + 1) * H)
        src_h = lax.slice_in_dim(src, h * H, (h + 1) * H)
        tf, sf = gather(xpk, tgt_h, src_h)
        tf2 = tf.reshape(HB, D)
        sf2 = sf.reshape(HB, D)
        attn2 = lax.slice_in_dim(attn_value, h * H, (h + 1) * H).reshape(HB, 2)
        mlo, mhi = dense(tf2, sf2, attn2, o2, o2t,
                         gl1, gh1, gl2, gh2, bl1, bh1, bl2, bh2, bias2,
                         bd1a, bd1b, bd2a, bd2b)
        tgt3d = tgt_h.reshape(NS, EPT // CS, CS)
        agg, cnt = scatter(mlo.reshape(H, HD), mhi.reshape(H, HD), tgt3d,
                           zero_agg, zero_cnt, ones_rows)
        parts.append((agg, cnt))

    Bn = 2000
    out = pl.pallas_call(
        _combine_body,
        grid=(N // Bn,),
        in_specs=[pl.BlockSpec((NC, Bn, HD), lambda i: (0, i, 0)),
                  pl.BlockSpec((NC, Bn, CW), lambda i: (0, i, 0))],
        out_specs=pl.BlockSpec((Bn, D), lambda i: (i, 0)),
        out_shape=jax.ShapeDtypeStruct((N, D), jnp.float32),
    )(parts[0][0], parts[0][1])
    return out
